# baseline (speedup 1.0000x reference)
import functools

import jax
import jax.numpy as jnp
from jax import lax
from jax.experimental import pallas as pl
from jax.experimental.pallas import tpu as pltpu
from jax.experimental.pallas import tpu_sc as plsc

HIDDEN = 64
B = 16384
NC = 2
NS = 16
NW = NC * NS
BPW = B // NW
QUAD = 1
NQ = BPW // QUAD
NBANK = 12
COLS = 256


def _make_kernel():
    mesh = plsc.VectorSubcoreMesh(core_axis_name="c", subcore_axis_name="s")

    @functools.partial(
        pl.kernel,
        mesh=mesh,
        out_type=jax.ShapeDtypeStruct((HIDDEN, B), jnp.float32),
        scratch_types=[
            pltpu.VMEM((BPW + 16,), jnp.int32),
            pltpu.VMEM((NBANK, QUAD, HIDDEN, 128), jnp.float32),
            pltpu.VMEM((HIDDEN, COLS), jnp.float32),
            pltpu.SemaphoreType.DMA((NBANK,)),
        ],
        compiler_params=pltpu.CompilerParams(needs_layout_passes=False),
    )
    def emb(idx_hbm, tblt_hbm, outt_hbm, idx_v, blocks_v, cols_v, sems):
        wid = lax.axis_index("s") * NC + lax.axis_index("c")
        pltpu.sync_copy(idx_hbm.at[wid], idx_v.at[pl.ds(0, BPW)])
        iota = lax.broadcasted_iota(jnp.int32, (16,), 0)

        def fetch(q, bank):
            vec = idx_v[pl.ds(q * QUAD, 16)]
            for j in range(QUAD):
                c = vec[j]
                base = pl.multiple_of((c >> 7) << 7, 128)
                pltpu.async_copy(
                    tblt_hbm.at[:, pl.ds(base, 128)],
                    blocks_v.at[bank, j],
                    sems.at[bank],
                )

        def extract(q, bank):
            vec = idx_v[pl.ds(q * QUAD, 16)]
            for j in range(QUAD):
                c = vec[j]
                pltpu.make_async_copy(
                    tblt_hbm.at[:, pl.ds(0, 128)],
                    blocks_v.at[bank, j],
                    sems.at[bank],
                ).wait()
                lane = jnp.broadcast_to(c & 127, (16,))
                ocol = jnp.broadcast_to(
                    (lax.rem(q, NQ // 2)) * QUAD + j, (16,)
                )
                for r in range(HIDDEN // 16):
                    rows = iota + (r * 16)
                    vals = plsc.load_gather(blocks_v.at[bank, j], [rows, lane])
                    plsc.store_scatter(cols_v, [rows, ocol], vals)

        for k in range(NBANK):
            fetch(k, k)

        def step(p, _):
            for k in range(NBANK):
                q = p * NBANK + k

                @pl.when(q < NQ)
                def _():
                    extract(q, k)

                    @pl.when(q + NBANK < NQ)
                    def _():
                        fetch(q + NBANK, k)

                    @pl.when(q == NQ // 2 - 1)
                    def _():
                        pltpu.sync_copy(
                            cols_v, outt_hbm.at[:, pl.ds(wid * BPW, COLS)]
                        )

                    @pl.when(q == NQ - 1)
                    def _():
                        pltpu.sync_copy(
                            cols_v,
                            outt_hbm.at[:, pl.ds(wid * BPW + COLS, COLS)],
                        )

            return ()

        lax.fori_loop(0, (NQ + NBANK - 1) // NBANK, step, ())

    return emb


_emb = _make_kernel()


def kernel(labels, table):
    idx = labels.astype(jnp.int32).reshape(NW, BPW)
    outt = _emb(idx, table.T)
    return outt.T


# speedup vs baseline: 1.0307x; 1.0307x over previous
import functools

import jax
import jax.numpy as jnp
from jax import lax
from jax.experimental import pallas as pl
from jax.experimental.pallas import tpu as pltpu
from jax.experimental.pallas import tpu_sc as plsc

HIDDEN = 64
B = 16384
NC = 2
NS = 16
NW = NC * NS
BPW = B // NW
QUAD = 2
NQ = BPW // QUAD
NBANK = 6
COLS = 256


def _make_kernel():
    mesh = plsc.VectorSubcoreMesh(core_axis_name="c", subcore_axis_name="s")

    @functools.partial(
        pl.kernel,
        mesh=mesh,
        out_type=jax.ShapeDtypeStruct((HIDDEN, B), jnp.float32),
        scratch_types=[
            pltpu.VMEM((BPW + 16,), jnp.int32),
            pltpu.VMEM((NBANK, QUAD, HIDDEN, 128), jnp.float32),
            pltpu.VMEM((HIDDEN, COLS), jnp.float32),
            pltpu.SemaphoreType.DMA((NBANK,)),
        ],
        compiler_params=pltpu.CompilerParams(needs_layout_passes=False),
    )
    def emb(idx_hbm, tblt_hbm, outt_hbm, idx_v, blocks_v, cols_v, sems):
        wid = lax.axis_index("s") * NC + lax.axis_index("c")
        pltpu.sync_copy(idx_hbm.at[wid], idx_v.at[pl.ds(0, BPW)])
        iota = lax.broadcasted_iota(jnp.int32, (16,), 0)

        def fetch(q, bank):
            vec = idx_v[pl.ds(q * QUAD, 16)]
            for j in range(QUAD):
                c = vec[j]
                base = pl.multiple_of((c >> 7) << 7, 128)
                pltpu.async_copy(
                    tblt_hbm.at[:, pl.ds(base, 128)],
                    blocks_v.at[bank, j],
                    sems.at[bank],
                )

        def extract(q, bank):
            vec = idx_v[pl.ds(q * QUAD, 16)]
            for j in range(QUAD):
                c = vec[j]
                pltpu.make_async_copy(
                    tblt_hbm.at[:, pl.ds(0, 128)],
                    blocks_v.at[bank, j],
                    sems.at[bank],
                ).wait()
                lane = jnp.broadcast_to(c & 127, (16,))
                ocol = jnp.broadcast_to(
                    (lax.rem(q, NQ // 2)) * QUAD + j, (16,)
                )
                for r in range(HIDDEN // 16):
                    rows = iota + (r * 16)
                    vals = plsc.load_gather(blocks_v.at[bank, j], [rows, lane])
                    plsc.store_scatter(cols_v, [rows, ocol], vals)

        for k in range(NBANK):
            fetch(k, k)

        def step(p, _):
            for k in range(NBANK):
                q = p * NBANK + k

                @pl.when(q < NQ)
                def _():
                    extract(q, k)

                    @pl.when(q + NBANK < NQ)
                    def _():
                        fetch(q + NBANK, k)

                    @pl.when(q == NQ // 2 - 1)
                    def _():
                        pltpu.sync_copy(
                            cols_v, outt_hbm.at[:, pl.ds(wid * BPW, COLS)]
                        )

                    @pl.when(q == NQ - 1)
                    def _():
                        pltpu.sync_copy(
                            cols_v,
                            outt_hbm.at[:, pl.ds(wid * BPW + COLS, COLS)],
                        )

            return ()

        lax.fori_loop(0, (NQ + NBANK - 1) // NBANK, step, ())

    return emb


_emb = _make_kernel()


def kernel(labels, table):
    idx = labels.astype(jnp.int32).reshape(NW, BPW)
    outt = _emb(idx, table.T)
    return outt.T
